# baseline (speedup 1.0000x reference)
import jax
import jax.numpy as jnp
from jax.experimental import pallas as pl
from jax.experimental.pallas import tpu as pltpu

B, H, S, E = 8, 16, 2048, 128
_NCHUNK = 8


def _kvcache_kernel(n_ref, k_t, v_t, k_cache, v_cache, out_k, out_v,
                    bulk_sems, row_sems):
    for i in range(_NCHUNK):
        pltpu.make_async_copy(k_cache.at[i], out_k.at[i], bulk_sems.at[i]).start()
        pltpu.make_async_copy(v_cache.at[i], out_v.at[i],
                              bulk_sems.at[_NCHUNK + i]).start()
    for i in range(_NCHUNK):
        pltpu.make_async_copy(k_cache.at[i], out_k.at[i], bulk_sems.at[i]).wait()
        pltpu.make_async_copy(v_cache.at[i], out_v.at[i],
                              bulk_sems.at[_NCHUNK + i]).wait()
    slot = n_ref[0] + 1
    k_row = pltpu.make_async_copy(k_t, out_k.at[:, :, pl.ds(slot, 1), :],
                                  row_sems.at[0])
    v_row = pltpu.make_async_copy(v_t, out_v.at[:, :, pl.ds(slot, 1), :],
                                  row_sems.at[1])
    k_row.start()
    v_row.start()
    k_row.wait()
    v_row.wait()


def kernel(k_t, v_t, k_cache, v_cache, n_cached):
    n_arr = jnp.asarray(n_cached, jnp.int32).reshape(1)
    hbm = pl.BlockSpec(memory_space=pltpu.MemorySpace.HBM)
    return pl.pallas_call(
        _kvcache_kernel,
        out_shape=(jax.ShapeDtypeStruct(k_cache.shape, k_cache.dtype),
                   jax.ShapeDtypeStruct(v_cache.shape, v_cache.dtype)),
        in_specs=[pl.BlockSpec(memory_space=pltpu.MemorySpace.SMEM),
                  hbm, hbm, hbm, hbm],
        out_specs=(hbm, hbm),
        scratch_shapes=[pltpu.SemaphoreType.DMA((2 * _NCHUNK,)),
                        pltpu.SemaphoreType.DMA((2,))],
    )(n_arr, k_t, v_t, k_cache, v_cache)


# speedup vs baseline: 43.4283x; 43.4283x over previous
import jax
import jax.numpy as jnp
from jax.experimental import pallas as pl
from jax.experimental.pallas import tpu as pltpu

B, H, S, E = 8, 16, 2048, 128


def _kvcache_kernel(n_ref, k_t, v_t, k_cache, v_cache, out_k, out_v):
    out_k[...] = k_cache[...]
    out_v[...] = v_cache[...]
    slot = n_ref[0] + 1
    out_k[0, 0, pl.ds(slot, 1), :] = k_t[0, 0, :, :]
    out_v[0, 0, pl.ds(slot, 1), :] = v_t[0, 0, :, :]


def kernel(k_t, v_t, k_cache, v_cache, n_cached):
    n_arr = jnp.asarray(n_cached, jnp.int32).reshape(1)
    cache_spec = pl.BlockSpec((1, 1, S, E), lambda b, h: (b, h, 0, 0))
    t_spec = pl.BlockSpec((1, 1, 1, E), lambda b, h: (b, h, 0, 0))
    return pl.pallas_call(
        _kvcache_kernel,
        grid=(B, H),
        out_shape=(jax.ShapeDtypeStruct(k_cache.shape, k_cache.dtype),
                   jax.ShapeDtypeStruct(v_cache.shape, v_cache.dtype)),
        in_specs=[pl.BlockSpec(memory_space=pltpu.MemorySpace.SMEM),
                  t_spec, t_spec, cache_spec, cache_spec],
        out_specs=(cache_spec, cache_spec),
    )(n_arr, k_t, v_t, k_cache, v_cache)


# speedup vs baseline: 48.6338x; 1.1199x over previous
import jax
import jax.numpy as jnp
from jax.experimental import pallas as pl
from jax.experimental.pallas import tpu as pltpu

B, H, S, E = 8, 16, 2048, 128


_HB = 4


def _kvcache_kernel(n_ref, k_t, v_t, k_cache, v_cache, out_k, out_v):
    out_k[...] = k_cache[...]
    out_v[...] = v_cache[...]
    slot = n_ref[0] + 1
    out_k[0, :, pl.ds(slot, 1), :] = k_t[0, :, :, :]
    out_v[0, :, pl.ds(slot, 1), :] = v_t[0, :, :, :]


def kernel(k_t, v_t, k_cache, v_cache, n_cached):
    n_arr = jnp.asarray(n_cached, jnp.int32).reshape(1)
    cache_spec = pl.BlockSpec((1, _HB, S, E), lambda b, h: (b, h, 0, 0))
    t_spec = pl.BlockSpec((1, _HB, 1, E), lambda b, h: (b, h, 0, 0))
    return pl.pallas_call(
        _kvcache_kernel,
        grid=(B, H // _HB),
        out_shape=(jax.ShapeDtypeStruct(k_cache.shape, k_cache.dtype),
                   jax.ShapeDtypeStruct(v_cache.shape, v_cache.dtype)),
        in_specs=[pl.BlockSpec(memory_space=pltpu.MemorySpace.SMEM),
                  t_spec, t_spec, cache_spec, cache_spec],
        out_specs=(cache_spec, cache_spec),
    )(n_arr, k_t, v_t, k_cache, v_cache)
